# baseline (speedup 1.0000x reference)
import jax
import jax.numpy as jnp
from jax import lax
from jax.experimental import pallas as pl

DIM = 4096
NUM_EXPERTS = 64
TOP_K = 8
TOKENS = 16384
CAPACITY_FACTOR = 1.0

BLOCK_T = 1024


SUB_T = 256


def _router_kernel(x_ref, wt_ref, b_ref, logits_ref, idx_ref, wts_ref, mask_ref):
    wt = wt_ref[...]
    b = b_ref[...]
    for c in range(BLOCK_T // SUB_T):
        sl = pl.ds(c * SUB_T, SUB_T)
        x = x_ref[sl, :]
        s = jnp.sum(x.reshape(SUB_T, 64, 64), axis=1)
        logits_ref[sl, :] = s
        mask_ref[sl, :] = s
        wts_ref[sl, :] = s[:, :8]
        idx_ref[sl, :] = jnp.zeros((SUB_T, TOP_K), jnp.int32)


def kernel(x, W, b):
    wt = W.T
    b2 = b.reshape(1, NUM_EXPERTS)
    grid = (TOKENS // BLOCK_T,)
    logits, idx, wts, mask = pl.pallas_call(
        _router_kernel,
        grid=grid,
        in_specs=[
            pl.BlockSpec((BLOCK_T, DIM), lambda i: (i, 0)),
            pl.BlockSpec((DIM, NUM_EXPERTS), lambda i: (0, 0)),
            pl.BlockSpec((1, NUM_EXPERTS), lambda i: (0, 0)),
        ],
        out_specs=[
            pl.BlockSpec((BLOCK_T, NUM_EXPERTS), lambda i: (i, 0)),
            pl.BlockSpec((BLOCK_T, TOP_K), lambda i: (i, 0)),
            pl.BlockSpec((BLOCK_T, TOP_K), lambda i: (i, 0)),
            pl.BlockSpec((BLOCK_T, NUM_EXPERTS), lambda i: (i, 0)),
        ],
        out_shape=[
            jax.ShapeDtypeStruct((TOKENS, NUM_EXPERTS), jnp.float32),
            jax.ShapeDtypeStruct((TOKENS, TOP_K), jnp.int32),
            jax.ShapeDtypeStruct((TOKENS, TOP_K), jnp.float32),
            jax.ShapeDtypeStruct((TOKENS, NUM_EXPERTS), jnp.float32),
        ],
    )(x, wt, b2)
    capacity = min(TOKENS, int(CAPACITY_FACTOR * TOKENS / NUM_EXPERTS * TOP_K))
    return (logits, idx, wts, mask, jnp.int32(capacity))


# speedup vs baseline: 1.4390x; 1.4390x over previous
import jax
import jax.numpy as jnp
from jax import lax
from jax.experimental import pallas as pl
from jax.experimental.pallas import tpu as pltpu

DIM = 4096
NUM_EXPERTS = 64
TOP_K = 8
TOKENS = 16384
CAPACITY_FACTOR = 1.0

BLOCK_T = 1024


SUB_T = 256


def _router_kernel(x_ref, wt_ref, b_ref, logits_ref, idx_ref, wts_ref, mask_ref):
    wt = wt_ref[...]
    b = b_ref[...]
    for c in range(BLOCK_T // SUB_T):
        sl = pl.ds(c * SUB_T, SUB_T)
        x = x_ref[sl, :]
        logits = lax.dot_general(
            x, wt, (((1,), (0,)), ((), ())), preferred_element_type=jnp.float32
        ) + b
        logits_ref[sl, :] = logits

        iota_f = lax.broadcasted_iota(jnp.int32, logits.shape, 1).astype(jnp.float32)
        work = logits
        vals = []
        idxs = []
        for _ in range(TOP_K):
            m = jnp.max(work, axis=1, keepdims=True)
            cand = jnp.where(work == m, iota_f, float(NUM_EXPERTS))
            idx_f = jnp.min(cand, axis=1, keepdims=True)
            work = jnp.where(iota_f == idx_f, -jnp.inf, work)
            vals.append(m)
            idxs.append(idx_f)
        mask_ref[sl, :] = (work == -jnp.inf).astype(jnp.float32)

        tv = jnp.concatenate(vals, axis=1)
        ti = jnp.concatenate(idxs, axis=1)
        e = jnp.exp(tv - tv[:, 0:1])
        wts_ref[sl, :] = e / jnp.sum(e, axis=1, keepdims=True)
        idx_ref[sl, :] = ti.astype(jnp.int32)


def kernel(x, W, b):
    wt = W.T
    b2 = b.reshape(1, NUM_EXPERTS)
    grid = (TOKENS // BLOCK_T,)
    logits, idx, wts, mask = pl.pallas_call(
        _router_kernel,
        grid=grid,
        in_specs=[
            pl.BlockSpec((BLOCK_T, DIM), lambda i: (i, 0)),
            pl.BlockSpec((DIM, NUM_EXPERTS), lambda i: (0, 0)),
            pl.BlockSpec((1, NUM_EXPERTS), lambda i: (0, 0)),
        ],
        out_specs=[
            pl.BlockSpec((BLOCK_T, NUM_EXPERTS), lambda i: (i, 0)),
            pl.BlockSpec((BLOCK_T, TOP_K), lambda i: (i, 0)),
            pl.BlockSpec((BLOCK_T, TOP_K), lambda i: (i, 0)),
            pl.BlockSpec((BLOCK_T, NUM_EXPERTS), lambda i: (i, 0)),
        ],
        out_shape=[
            jax.ShapeDtypeStruct((TOKENS, NUM_EXPERTS), jnp.float32),
            jax.ShapeDtypeStruct((TOKENS, TOP_K), jnp.int32),
            jax.ShapeDtypeStruct((TOKENS, TOP_K), jnp.float32),
            jax.ShapeDtypeStruct((TOKENS, NUM_EXPERTS), jnp.float32),
        ],
        compiler_params=pltpu.CompilerParams(
            dimension_semantics=("parallel",),
        ),
    )(x, wt, b2)
    capacity = min(TOKENS, int(CAPACITY_FACTOR * TOKENS / NUM_EXPERTS * TOP_K))
    return (logits, idx, wts, mask, jnp.int32(capacity))


# speedup vs baseline: 1.4454x; 1.0045x over previous
import jax
import jax.numpy as jnp
from jax import lax
from jax.experimental import pallas as pl
from jax.experimental.pallas import tpu as pltpu

DIM = 4096
NUM_EXPERTS = 64
TOP_K = 8
TOKENS = 16384
CAPACITY_FACTOR = 1.0

BLOCK_T = 1024


SUB_T = 256


def _router_kernel(xa_ref, xb_ref, wt_ref, b_ref, logits_ref, idx_ref, wts_ref,
                   mask_ref):
    wt = wt_ref[...]
    b = b_ref[...]
    half = BLOCK_T // 2
    for c in range(BLOCK_T // SUB_T):
        sl = pl.ds(c * SUB_T, SUB_T)
        if c < (BLOCK_T // SUB_T) // 2:
            x = xa_ref[pl.ds(c * SUB_T, SUB_T), :]
        else:
            x = xb_ref[pl.ds(c * SUB_T - half, SUB_T), :]
        logits = lax.dot_general(
            x, wt, (((1,), (0,)), ((), ())), preferred_element_type=jnp.float32
        ) + b
        logits_ref[sl, :] = logits

        iota_f = lax.broadcasted_iota(jnp.int32, logits.shape, 1).astype(jnp.float32)
        work = logits
        vals = []
        idxs = []
        for _ in range(TOP_K):
            m = jnp.max(work, axis=1, keepdims=True)
            cand = jnp.where(work == m, iota_f, float(NUM_EXPERTS))
            idx_f = jnp.min(cand, axis=1, keepdims=True)
            work = jnp.where(iota_f == idx_f, -jnp.inf, work)
            vals.append(m)
            idxs.append(idx_f)
        mask_ref[sl, :] = (work == -jnp.inf).astype(jnp.float32)

        tv = jnp.concatenate(vals, axis=1)
        ti = jnp.concatenate(idxs, axis=1)
        e = jnp.exp(tv - tv[:, 0:1])
        wts_ref[sl, :] = e / jnp.sum(e, axis=1, keepdims=True)
        idx_ref[sl, :] = ti.astype(jnp.int32)


def kernel(x, W, b):
    wt = W.T
    b2 = b.reshape(1, NUM_EXPERTS)
    grid = (TOKENS // BLOCK_T,)
    logits, idx, wts, mask = pl.pallas_call(
        _router_kernel,
        grid=grid,
        in_specs=[
            pl.BlockSpec((BLOCK_T // 2, DIM), lambda i: (2 * i, 0)),
            pl.BlockSpec((BLOCK_T // 2, DIM), lambda i: (2 * i + 1, 0)),
            pl.BlockSpec((DIM, NUM_EXPERTS), lambda i: (0, 0)),
            pl.BlockSpec((1, NUM_EXPERTS), lambda i: (0, 0)),
        ],
        out_specs=[
            pl.BlockSpec((BLOCK_T, NUM_EXPERTS), lambda i: (i, 0)),
            pl.BlockSpec((BLOCK_T, TOP_K), lambda i: (i, 0)),
            pl.BlockSpec((BLOCK_T, TOP_K), lambda i: (i, 0)),
            pl.BlockSpec((BLOCK_T, NUM_EXPERTS), lambda i: (i, 0)),
        ],
        out_shape=[
            jax.ShapeDtypeStruct((TOKENS, NUM_EXPERTS), jnp.float32),
            jax.ShapeDtypeStruct((TOKENS, TOP_K), jnp.int32),
            jax.ShapeDtypeStruct((TOKENS, TOP_K), jnp.float32),
            jax.ShapeDtypeStruct((TOKENS, NUM_EXPERTS), jnp.float32),
        ],
        compiler_params=pltpu.CompilerParams(
            dimension_semantics=("parallel",),
        ),
    )(x, x, wt, b2)
    capacity = min(TOKENS, int(CAPACITY_FACTOR * TOKENS / NUM_EXPERTS * TOP_K))
    return (logits, idx, wts, mask, jnp.int32(capacity))
